# baseline (speedup 1.0000x reference)
import functools

import jax
import jax.numpy as jnp
from jax.experimental import pallas as pl
from jax.experimental.pallas import tpu as pltpu

GH, GW, NA, CIN, CMID = 32, 32, 9, 768, 256
NPOS = GH * GW
N = NPOS * NA
BK = 128
NB = N // BK
MAX_POST = 2000
NOUT = 2048
NOB = NOUT // BK
IOU_T = 0.7


def _heads_kernel(x_ref, wb_ref, bb_ref, wc_ref, bc_ref, wr_ref, br_ref,
                  valid_ref, cls_ref, reg_ref, sm_ref):
    h = jnp.maximum(
        jnp.dot(x_ref[:], wb_ref[:], preferred_element_type=jnp.float32)
        + bb_ref[:], 0.0)
    logits = jnp.dot(h, wc_ref[:], preferred_element_type=jnp.float32) + bc_ref[:]
    cls = jax.nn.sigmoid(logits)
    reg = jnp.dot(h, wr_ref[:], preferred_element_type=jnp.float32) + br_ref[:]
    cls_ref[:] = cls
    reg_ref[:] = reg
    sm_ref[:] = jnp.where(valid_ref[:] > 0.0, cls, -jnp.inf)


def _nms_kernel(s_row_ref, s_col_ref, a4_ref, out_ref,
                rank_c, sP, keep_r, sup_r, destk_c, aux_r):
    f32 = jnp.float32
    a0 = a4_ref[0:1, :]
    a1 = a4_ref[1:2, :]
    a2 = a4_ref[2:3, :]
    a3 = a4_ref[3:4, :]
    c0 = a0 + a0 * a2
    c1 = a1 + a1 * a3
    w0 = a2 * jnp.exp(a2)
    w1 = a3 * jnp.exp(a3)
    P0 = c0 - w0 * 0.5
    P1 = c1 - w1 * 0.5
    P2 = c0 + w0 * 0.5
    P3 = c1 + w1 * 0.5
    P4 = jnp.concatenate([P0, P1, P2, P3], axis=0)

    iota_c = jax.lax.broadcasted_iota(jnp.int32, (BK, 1), 0)
    iota_r = jax.lax.broadcasted_iota(jnp.int32, (1, BK), 1)

    aux_r[:] = jnp.zeros((1, N), f32)
    su_bool = iota_c < iota_r

    def rank_body(bi, carry):
        base = bi * BK
        sc = s_col_ref[pl.ds(base, BK), :]
        srb = s_row_ref[0:1, pl.ds(base, BK)]
        diag = (srb > sc) | ((srb == sc) & su_bool)
        acc0 = jnp.sum(diag.astype(f32), axis=1, keepdims=True)

        def pair_body(bj, acc):
            cb = bj * BK
            sj = s_row_ref[0:1, pl.ds(cb, BK)]
            acc = acc + jnp.sum((sj >= sc).astype(f32), axis=1, keepdims=True)
            rev = jnp.sum((sj < sc).astype(f32), axis=0, keepdims=True)
            aux_r[0:1, pl.ds(cb, BK)] = aux_r[0:1, pl.ds(cb, BK)] + rev
            return acc

        acc = jax.lax.fori_loop(bi + 1, NB, pair_body, acc0, unroll=False)
        rank_c[pl.ds(base, BK), :] = acc
        return carry

    jax.lax.fori_loop(0, NB, rank_body, 0, unroll=False)

    def rank_fold(b, carry):
        base = b * BK
        rank_c[pl.ds(base, BK), :] = rank_c[pl.ds(base, BK), :] + jnp.transpose(
            aux_r[0:1, pl.ds(base, BK)], (1, 0))
        return carry

    jax.lax.fori_loop(0, NB, rank_fold, 0, unroll=False)

    rank_all = rank_c[:]

    def sort_body(db, carry):
        q = (iota_r + db * BK).astype(f32)
        oh = (rank_all == q).astype(f32)
        blk = jnp.dot(P4, oh, preferred_element_type=f32,
                      precision=jax.lax.Precision.HIGHEST)
        sP[:, pl.ds(db * BK, BK)] = blk
        return carry

    jax.lax.fori_loop(0, NB, sort_body, 0, unroll=False)

    sup_r[:] = jnp.zeros((1, N), f32)
    strict_upper = (iota_c < iota_r).astype(f32)

    def nms_body(b, carry):
        base = b * BK
        y1r = sP[0:1, pl.ds(base, BK)]
        x1r = sP[1:2, pl.ds(base, BK)]
        y2r = sP[2:3, pl.ds(base, BK)]
        x2r = sP[3:4, pl.ds(base, BK)]
        ar = (y2r - y1r) * (x2r - x1r)
        y1c = jnp.transpose(y1r, (1, 0))
        x1c = jnp.transpose(x1r, (1, 0))
        y2c = jnp.transpose(y2r, (1, 0))
        x2c = jnp.transpose(x2r, (1, 0))
        ac = jnp.transpose(ar, (1, 0))
        supc = jnp.transpose(sup_r[0:1, pl.ds(base, BK)], (1, 0))
        ih = jnp.minimum(y2c, y2r) - jnp.maximum(y1c, y1r)
        iw = jnp.minimum(x2c, x2r) - jnp.maximum(x1c, x1r)
        inter = jnp.maximum(ih, 0.0) * jnp.maximum(iw, 0.0)
        Sb = (inter > IOU_T * (ac + ar - inter + 1e-9)).astype(f32) * strict_upper
        init_r = jnp.transpose((supc == 0.0).astype(f32), (1, 0))

        def fcond(st):
            it, changed, _ = st
            return (it < BK + 2) & changed

        def fbody(st):
            it, _, k = st
            su = jnp.dot(k, Sb, preferred_element_type=f32)
            nk = init_r * (su == 0.0).astype(f32)
            return it + 1, jnp.any(nk != k), nk

        _, _, k_row = jax.lax.while_loop(fcond, fbody, (0, True, init_r))
        keep_r[0:1, pl.ds(base, BK)] = k_row
        k_col = jnp.transpose(k_row, (1, 0))
        tac = ac + 1e-9

        def chunk_body(bj, c):
            cb = bj * BK
            jy1 = sP[0:1, pl.ds(cb, BK)]
            jx1 = sP[1:2, pl.ds(cb, BK)]
            jy2 = sP[2:3, pl.ds(cb, BK)]
            jx2 = sP[3:4, pl.ds(cb, BK)]
            jar = (jy2 - jy1) * (jx2 - jx1)
            cih = jnp.minimum(y2c, jy2) - jnp.maximum(y1c, jy1)
            ciw = jnp.minimum(x2c, jx2) - jnp.maximum(x1c, jx1)
            cin_ = jnp.maximum(cih, 0.0) * jnp.maximum(ciw, 0.0)
            csup = (cin_ > IOU_T * (tac + jar - cin_)).astype(f32) * k_col
            news = (jnp.sum(csup, axis=0, keepdims=True) > 0.0).astype(f32)
            sup_r[0:1, pl.ds(cb, BK)] = jnp.maximum(sup_r[0:1, pl.ds(cb, BK)], news)
            return c

        jax.lax.fori_loop(b + 1, NB, chunk_body, 0, unroll=False)
        return carry

    jax.lax.fori_loop(0, NB, nms_body, 0, unroll=False)

    def dest_body(b, nkept):
        krb = keep_r[0:1, pl.ds(b * BK, BK)]
        excl = jnp.dot(krb, strict_upper, preferred_element_type=f32)
        destk = jnp.where(krb > 0.0, excl + nkept, -1.0)
        destk_c[pl.ds(b * BK, BK), :] = jnp.transpose(destk, (1, 0))
        return nkept + jnp.sum(krb)

    nkept = jax.lax.fori_loop(0, NB, dest_body, 0.0, unroll=False)

    destk_all = destk_c[:]
    sb0 = sP[:, 0:1]

    def out_body(ob, carry):
        p = (iota_r + ob * BK).astype(f32)
        oh = (destk_all == p).astype(f32)
        blk = jnp.dot(sP[:, :], oh, preferred_element_type=f32,
                      precision=jax.lax.Precision.HIGHEST)
        blk = blk + (p >= nkept).astype(f32) * sb0
        out_ref[:, pl.ds(ob * BK, BK)] = blk
        return carry

    jax.lax.fori_loop(0, NOB, out_body, 0, unroll=False)


@functools.partial(jax.jit, static_argnames=())
def _run(feats, ancs, ancs_valid, W_b, b_b, W_cls, b_cls, W_reg, b_reg):
    x = feats.reshape(NPOS, CIN)
    valid2d = ancs_valid.reshape(NPOS, NA)
    cls2d, reg2d, sm2d = pl.pallas_call(
        _heads_kernel,
        out_shape=(
            jax.ShapeDtypeStruct((NPOS, NA), jnp.float32),
            jax.ShapeDtypeStruct((NPOS, NA * 4), jnp.float32),
            jax.ShapeDtypeStruct((NPOS, NA), jnp.float32),
        ),
    )(x, W_b, b_b.reshape(1, CMID), W_cls, b_cls.reshape(1, NA),
      W_reg, b_reg.reshape(1, NA * 4), valid2d)

    s_row = sm2d.reshape(1, N)
    s_col = sm2d.reshape(N, 1)
    a4 = ancs.reshape(N, 4).T

    out4 = pl.pallas_call(
        _nms_kernel,
        out_shape=jax.ShapeDtypeStruct((4, NOUT), jnp.float32),
        scratch_shapes=[
            pltpu.VMEM((N, 1), jnp.float32),
            pltpu.VMEM((4, N), jnp.float32),
            pltpu.VMEM((1, N), jnp.float32),
            pltpu.VMEM((1, N), jnp.float32),
            pltpu.VMEM((N, 1), jnp.float32),
            pltpu.VMEM((1, N), jnp.float32),
        ],
    )(s_row, s_col, a4)

    cls_pred = cls2d.reshape(1, GH, GW, NA)
    reg_pred = reg2d.reshape(1, GH, GW, NA, 4)
    boxes = out4.T[:MAX_POST, :]
    return (cls_pred, reg_pred, boxes)


def kernel(feats, ancs, ancs_valid, W_b, b_b, W_cls, b_cls, W_reg, b_reg):
    return _run(feats, ancs, ancs_valid, W_b, b_b, W_cls, b_cls, W_reg, b_reg)


# speedup vs baseline: 1.2416x; 1.2416x over previous
import functools

import jax
import jax.numpy as jnp
from jax.experimental import pallas as pl
from jax.experimental.pallas import tpu as pltpu

GH, GW, NA, CIN, CMID = 32, 32, 9, 768, 256
NPOS = GH * GW
N = NPOS * NA
BK = 128
NB = N // BK
MAX_POST = 2000
NOUT = 2048
NOB = NOUT // BK
IOU_T = 0.7


def _heads_kernel(x_ref, wb_ref, bb_ref, wc_ref, bc_ref, wr_ref, br_ref,
                  valid_ref, cls_ref, reg_ref, sm_ref):
    h = jnp.maximum(
        jnp.dot(x_ref[:], wb_ref[:], preferred_element_type=jnp.float32)
        + bb_ref[:], 0.0)
    logits = jnp.dot(h, wc_ref[:], preferred_element_type=jnp.float32) + bc_ref[:]
    cls = jax.nn.sigmoid(logits)
    reg = jnp.dot(h, wr_ref[:], preferred_element_type=jnp.float32) + br_ref[:]
    cls_ref[:] = cls
    reg_ref[:] = reg
    sm_ref[:] = jnp.where(valid_ref[:] > 0.0, cls, -jnp.inf)


def _nms_kernel(s_row_ref, s_col_ref, a4_ref, out_ref,
                rank_c, sP, keep_r, sup_r, destk_c):
    f32 = jnp.float32
    a0 = a4_ref[0:1, :]
    a1 = a4_ref[1:2, :]
    a2 = a4_ref[2:3, :]
    a3 = a4_ref[3:4, :]
    c0 = a0 + a0 * a2
    c1 = a1 + a1 * a3
    w0 = a2 * jnp.exp(a2)
    w1 = a3 * jnp.exp(a3)
    P0 = c0 - w0 * 0.5
    P1 = c1 - w1 * 0.5
    P2 = c0 + w0 * 0.5
    P3 = c1 + w1 * 0.5
    P4 = jnp.concatenate([P0, P1, P2, P3], axis=0)

    iota_c = jax.lax.broadcasted_iota(jnp.int32, (BK, 1), 0)
    iota_r = jax.lax.broadcasted_iota(jnp.int32, (1, BK), 1)

    idx_row = jax.lax.broadcasted_iota(jnp.int32, (1, N), 1)
    s_row = s_row_ref[:]

    def rank_body(bi, carry):
        base = bi * BK
        sc = s_col_ref[pl.ds(base, BK), :]
        ic = iota_c + base
        lex = (s_row > sc) | ((s_row == sc) & (idx_row > ic))
        rank_c[pl.ds(base, BK), :] = jnp.sum(lex.astype(f32), axis=1,
                                             keepdims=True)
        return carry

    jax.lax.fori_loop(0, NB, rank_body, 0, unroll=False)

    rank_all = rank_c[:]

    def sort_body(db, carry):
        q = (iota_r + db * BK).astype(f32)
        oh = (rank_all == q).astype(f32)
        blk = jnp.dot(P4, oh, preferred_element_type=f32,
                      precision=jax.lax.Precision.HIGHEST)
        sP[:, pl.ds(db * BK, BK)] = blk
        return carry

    jax.lax.fori_loop(0, NB, sort_body, 0, unroll=False)

    sup_r[:] = jnp.zeros((1, N), f32)
    strict_upper = (iota_c < iota_r).astype(f32)

    def nms_body(b, carry):
        base = b * BK
        y1r = sP[0:1, pl.ds(base, BK)]
        x1r = sP[1:2, pl.ds(base, BK)]
        y2r = sP[2:3, pl.ds(base, BK)]
        x2r = sP[3:4, pl.ds(base, BK)]
        ar = (y2r - y1r) * (x2r - x1r)
        y1c = jnp.transpose(y1r, (1, 0))
        x1c = jnp.transpose(x1r, (1, 0))
        y2c = jnp.transpose(y2r, (1, 0))
        x2c = jnp.transpose(x2r, (1, 0))
        ac = jnp.transpose(ar, (1, 0))
        supc = jnp.transpose(sup_r[0:1, pl.ds(base, BK)], (1, 0))
        ih = jnp.minimum(y2c, y2r) - jnp.maximum(y1c, y1r)
        iw = jnp.minimum(x2c, x2r) - jnp.maximum(x1c, x1r)
        inter = jnp.maximum(ih, 0.0) * jnp.maximum(iw, 0.0)
        Sb = (inter > IOU_T * (ac + ar - inter + 1e-9)).astype(f32) * strict_upper
        init_r = jnp.transpose((supc == 0.0).astype(f32), (1, 0))

        def fcond(st):
            it, changed, _ = st
            return (it < BK + 2) & changed

        def fbody(st):
            it, _, k = st
            su = jnp.dot(k, Sb, preferred_element_type=f32)
            nk = init_r * (su == 0.0).astype(f32)
            return it + 1, jnp.any(nk != k), nk

        _, _, k_row = jax.lax.while_loop(fcond, fbody, (0, True, init_r))
        keep_r[0:1, pl.ds(base, BK)] = k_row
        k_col = jnp.transpose(k_row, (1, 0))
        tac = ac + 1e-9

        def chunk_body(bj, c):
            cb = bj * BK
            jy1 = sP[0:1, pl.ds(cb, BK)]
            jx1 = sP[1:2, pl.ds(cb, BK)]
            jy2 = sP[2:3, pl.ds(cb, BK)]
            jx2 = sP[3:4, pl.ds(cb, BK)]
            jar = (jy2 - jy1) * (jx2 - jx1)
            cih = jnp.minimum(y2c, jy2) - jnp.maximum(y1c, jy1)
            ciw = jnp.minimum(x2c, jx2) - jnp.maximum(x1c, jx1)
            cin_ = jnp.maximum(cih, 0.0) * jnp.maximum(ciw, 0.0)
            csup = (cin_ > IOU_T * (tac + jar - cin_)).astype(f32) * k_col
            news = (jnp.sum(csup, axis=0, keepdims=True) > 0.0).astype(f32)
            sup_r[0:1, pl.ds(cb, BK)] = jnp.maximum(sup_r[0:1, pl.ds(cb, BK)], news)
            return c

        jax.lax.fori_loop(b + 1, NB, chunk_body, 0, unroll=False)
        return carry

    jax.lax.fori_loop(0, NB, nms_body, 0, unroll=False)

    def dest_body(b, nkept):
        krb = keep_r[0:1, pl.ds(b * BK, BK)]
        excl = jnp.dot(krb, strict_upper, preferred_element_type=f32)
        destk = jnp.where(krb > 0.0, excl + nkept, -1.0)
        destk_c[pl.ds(b * BK, BK), :] = jnp.transpose(destk, (1, 0))
        return nkept + jnp.sum(krb)

    nkept = jax.lax.fori_loop(0, NB, dest_body, 0.0, unroll=False)

    destk_all = destk_c[:]
    sb0 = sP[:, 0:1]

    def out_body(ob, carry):
        p = (iota_r + ob * BK).astype(f32)
        oh = (destk_all == p).astype(f32)
        blk = jnp.dot(sP[:, :], oh, preferred_element_type=f32,
                      precision=jax.lax.Precision.HIGHEST)
        blk = blk + (p >= nkept).astype(f32) * sb0
        out_ref[:, pl.ds(ob * BK, BK)] = blk
        return carry

    jax.lax.fori_loop(0, NOB, out_body, 0, unroll=False)


@functools.partial(jax.jit, static_argnames=())
def _run(feats, ancs, ancs_valid, W_b, b_b, W_cls, b_cls, W_reg, b_reg):
    x = feats.reshape(NPOS, CIN)
    valid2d = ancs_valid.reshape(NPOS, NA)
    cls2d, reg2d, sm2d = pl.pallas_call(
        _heads_kernel,
        out_shape=(
            jax.ShapeDtypeStruct((NPOS, NA), jnp.float32),
            jax.ShapeDtypeStruct((NPOS, NA * 4), jnp.float32),
            jax.ShapeDtypeStruct((NPOS, NA), jnp.float32),
        ),
    )(x, W_b, b_b.reshape(1, CMID), W_cls, b_cls.reshape(1, NA),
      W_reg, b_reg.reshape(1, NA * 4), valid2d)

    s_row = sm2d.reshape(1, N)
    s_col = sm2d.reshape(N, 1)
    a4 = ancs.reshape(N, 4).T

    out4 = pl.pallas_call(
        _nms_kernel,
        out_shape=jax.ShapeDtypeStruct((4, NOUT), jnp.float32),
        scratch_shapes=[
            pltpu.VMEM((N, 1), jnp.float32),
            pltpu.VMEM((4, N), jnp.float32),
            pltpu.VMEM((1, N), jnp.float32),
            pltpu.VMEM((1, N), jnp.float32),
            pltpu.VMEM((N, 1), jnp.float32),
        ],
    )(s_row, s_col, a4)

    cls_pred = cls2d.reshape(1, GH, GW, NA)
    reg_pred = reg2d.reshape(1, GH, GW, NA, 4)
    boxes = out4.T[:MAX_POST, :]
    return (cls_pred, reg_pred, boxes)


def kernel(feats, ancs, ancs_valid, W_b, b_b, W_cls, b_cls, W_reg, b_reg):
    return _run(feats, ancs, ancs_valid, W_b, b_b, W_cls, b_cls, W_reg, b_reg)


# speedup vs baseline: 1.5429x; 1.2427x over previous
import functools

import jax
import jax.numpy as jnp
from jax import lax
from jax.experimental import pallas as pl
from jax.experimental.pallas import tpu as pltpu
from jax.experimental.pallas import tpu_sc as plsc

GH, GW, NA, CIN, CMID = 32, 32, 9, 768, 256
NPOS = GH * GW
N = NPOS * NA
BK = 128
NB = N // BK
MAX_POST = 2000
NOUT = 2048
NOB = NOUT // BK
IOU_T = 0.7
SC_NW = 32
BPW = N // SC_NW


def _heads_kernel(x_ref, wb_ref, bb_ref, wc_ref, bc_ref, wr_ref, br_ref,
                  valid_ref, cls_ref, reg_ref, sm_ref):
    h = jnp.maximum(
        jnp.dot(x_ref[:], wb_ref[:], preferred_element_type=jnp.float32)
        + bb_ref[:], 0.0)
    logits = jnp.dot(h, wc_ref[:], preferred_element_type=jnp.float32) + bc_ref[:]
    cls = jax.nn.sigmoid(logits)
    reg = jnp.dot(h, wr_ref[:], preferred_element_type=jnp.float32) + br_ref[:]
    cls_ref[:] = cls
    reg_ref[:] = reg
    sm_ref[:] = jnp.where(valid_ref[:] > 0.0, cls, -jnp.inf)


def _rank_kernel(s_row_ref, s_col_ref, a4_ref, rank_ref, p16_ref, p4s):
    f32 = jnp.float32
    a0 = a4_ref[0:1, :]
    a1 = a4_ref[1:2, :]
    a2 = a4_ref[2:3, :]
    a3 = a4_ref[3:4, :]
    c0 = a0 + a0 * a2
    c1 = a1 + a1 * a3
    w0 = a2 * jnp.exp(a2)
    w1 = a3 * jnp.exp(a3)
    P0 = c0 - w0 * 0.5
    P1 = c1 - w1 * 0.5
    P2 = c0 + w0 * 0.5
    P3 = c1 + w1 * 0.5
    p4s[:] = jnp.concatenate([P0, P1, P2, P3], axis=0)

    iota_c = jax.lax.broadcasted_iota(jnp.int32, (BK, 1), 0)
    idx_row = jax.lax.broadcasted_iota(jnp.int32, (1, N), 1)
    s_row = s_row_ref[:]
    zpad = jnp.zeros((BK, 124), f32)

    def body(bi, carry):
        base = bi * BK
        sc = s_col_ref[pl.ds(base, BK), :]
        ic = iota_c + base
        gt = (s_row > sc).astype(f32)
        tie = ((s_row == sc) & (idx_row > ic)).astype(f32)
        rk = jnp.sum(gt + tie, axis=1, keepdims=True)
        rank_ref[pl.ds(base, BK), :] = rk.astype(jnp.int32)
        blk = jnp.transpose(p4s[:, pl.ds(base, BK)], (1, 0))
        p16_ref[pl.ds(base, BK), :] = jnp.concatenate([blk, zpad], axis=1)
        return carry

    jax.lax.fori_loop(0, NB, body, 0, unroll=False)


def _permute_sc(p16_hbm, rank_hbm, out_hbm, idx_v, rows_v, sem):
    wid = lax.axis_index("s") * 2 + lax.axis_index("c")
    base = wid * BPW
    pltpu.sync_copy(rank_hbm.at[pl.ds(base, BPW)], idx_v)
    pltpu.sync_copy(p16_hbm.at[pl.ds(base, BPW)], rows_v)
    pltpu.async_copy(rows_v, out_hbm.at[idx_v], sem).wait()


def _nms_kernel(sb_ref, out_ref, sP, keep_r, sup_r, destk_c):
    f32 = jnp.float32
    iota_c = jax.lax.broadcasted_iota(jnp.int32, (BK, 1), 0)
    iota_r = jax.lax.broadcasted_iota(jnp.int32, (1, BK), 1)
    strict_upper = (iota_c < iota_r).astype(f32)

    def plane_body(b, carry):
        base = b * BK
        t = jnp.transpose(sb_ref[pl.ds(base, BK), :], (1, 0))
        sP[:, pl.ds(base, BK)] = t[0:4, :]
        return carry

    jax.lax.fori_loop(0, NB, plane_body, 0, unroll=False)
    sup_r[:] = jnp.zeros((1, N), f32)

    def nms_body(b, carry):
        base = b * BK
        y1c = sb_ref[pl.ds(base, BK), 0:1]
        x1c = sb_ref[pl.ds(base, BK), 1:2]
        y2c = sb_ref[pl.ds(base, BK), 2:3]
        x2c = sb_ref[pl.ds(base, BK), 3:4]
        ac = (y2c - y1c) * (x2c - x1c)
        y1r = sP[0:1, pl.ds(base, BK)]
        x1r = sP[1:2, pl.ds(base, BK)]
        y2r = sP[2:3, pl.ds(base, BK)]
        x2r = sP[3:4, pl.ds(base, BK)]
        ar = (y2r - y1r) * (x2r - x1r)
        supc = jnp.transpose(sup_r[0:1, pl.ds(base, BK)], (1, 0))
        ih = jnp.minimum(y2c, y2r) - jnp.maximum(y1c, y1r)
        iw = jnp.minimum(x2c, x2r) - jnp.maximum(x1c, x1r)
        inter = jnp.maximum(ih, 0.0) * jnp.maximum(iw, 0.0)
        Sb = (inter > IOU_T * (ac + ar - inter + 1e-9)).astype(f32) * strict_upper
        init_r = jnp.transpose((supc == 0.0).astype(f32), (1, 0))

        def fcond(st):
            it, changed, _ = st
            return (it < BK + 2) & changed

        def fbody(st):
            it, _, k = st
            su = jnp.dot(k, Sb, preferred_element_type=f32)
            nk = init_r * (su == 0.0).astype(f32)
            return it + 1, jnp.any(nk != k), nk

        _, _, k_row = jax.lax.while_loop(fcond, fbody, (0, True, init_r))
        keep_r[0:1, pl.ds(base, BK)] = k_row
        k_col = jnp.transpose(k_row, (1, 0))
        tac = ac + 1e-9

        def chunk_body(bj, c):
            cb = bj * BK
            jy1 = sP[0:1, pl.ds(cb, BK)]
            jx1 = sP[1:2, pl.ds(cb, BK)]
            jy2 = sP[2:3, pl.ds(cb, BK)]
            jx2 = sP[3:4, pl.ds(cb, BK)]
            jar = (jy2 - jy1) * (jx2 - jx1)
            cih = jnp.minimum(y2c, jy2) - jnp.maximum(y1c, jy1)
            ciw = jnp.minimum(x2c, jx2) - jnp.maximum(x1c, jx1)
            cin_ = jnp.maximum(cih, 0.0) * jnp.maximum(ciw, 0.0)
            csup = (cin_ > IOU_T * (tac + jar - cin_)).astype(f32) * k_col
            news = (jnp.sum(csup, axis=0, keepdims=True) > 0.0).astype(f32)
            sup_r[0:1, pl.ds(cb, BK)] = jnp.maximum(sup_r[0:1, pl.ds(cb, BK)], news)
            return c

        jax.lax.fori_loop(b + 1, NB, chunk_body, 0, unroll=False)
        return carry

    jax.lax.fori_loop(0, NB, nms_body, 0, unroll=False)

    def dest_body(b, nkept):
        krb = keep_r[0:1, pl.ds(b * BK, BK)]
        excl = jnp.dot(krb, strict_upper, preferred_element_type=f32)
        destk = jnp.where(krb > 0.0, excl + nkept, -1.0)
        destk_c[pl.ds(b * BK, BK), :] = jnp.transpose(destk, (1, 0))
        return nkept + jnp.sum(krb)

    nkept = jax.lax.fori_loop(0, NB, dest_body, 0.0, unroll=False)

    destk_all = destk_c[:]
    sb0 = sP[:, 0:1]

    def out_body(ob, carry):
        p = (iota_r + ob * BK).astype(f32)
        oh = (destk_all == p).astype(f32)
        blk = jnp.dot(sP[:, :], oh, preferred_element_type=f32,
                      precision=jax.lax.Precision.HIGHEST)
        blk = blk + (p >= nkept).astype(f32) * sb0
        out_ref[:, pl.ds(ob * BK, BK)] = blk
        return carry

    jax.lax.fori_loop(0, NOB, out_body, 0, unroll=False)


@functools.partial(jax.jit, static_argnames=())
def _run(feats, ancs, ancs_valid, W_b, b_b, W_cls, b_cls, W_reg, b_reg):
    x = feats.reshape(NPOS, CIN)
    valid2d = ancs_valid.reshape(NPOS, NA)
    cls2d, reg2d, sm2d = pl.pallas_call(
        _heads_kernel,
        out_shape=(
            jax.ShapeDtypeStruct((NPOS, NA), jnp.float32),
            jax.ShapeDtypeStruct((NPOS, NA * 4), jnp.float32),
            jax.ShapeDtypeStruct((NPOS, NA), jnp.float32),
        ),
    )(x, W_b, b_b.reshape(1, CMID), W_cls, b_cls.reshape(1, NA),
      W_reg, b_reg.reshape(1, NA * 4), valid2d)

    s_row = sm2d.reshape(1, N)
    s_col = sm2d.reshape(N, 1)
    a4 = ancs.reshape(N, 4).T

    rank2d, p16 = pl.pallas_call(
        _rank_kernel,
        out_shape=(
            jax.ShapeDtypeStruct((N, 1), jnp.int32),
            jax.ShapeDtypeStruct((N, 128), jnp.float32),
        ),
        scratch_shapes=[pltpu.VMEM((4, N), jnp.float32)],
    )(s_row, s_col, a4)

    mesh = plsc.VectorSubcoreMesh(core_axis_name="c", subcore_axis_name="s")
    sorted16 = pl.kernel(
        _permute_sc,
        mesh=mesh,
        out_type=jax.ShapeDtypeStruct((N, 128), jnp.float32),
        scratch_types=[
            pltpu.VMEM((BPW,), jnp.int32),
            pltpu.VMEM((BPW, 128), jnp.float32),
            pltpu.SemaphoreType.DMA,
        ],
    )(p16, rank2d.reshape(N))

    out4 = pl.pallas_call(
        _nms_kernel,
        out_shape=jax.ShapeDtypeStruct((4, NOUT), jnp.float32),
        scratch_shapes=[
            pltpu.VMEM((4, N), jnp.float32),
            pltpu.VMEM((1, N), jnp.float32),
            pltpu.VMEM((1, N), jnp.float32),
            pltpu.VMEM((N, 1), jnp.float32),
        ],
    )(sorted16)

    cls_pred = cls2d.reshape(1, GH, GW, NA)
    reg_pred = reg2d.reshape(1, GH, GW, NA, 4)
    boxes = out4.T[:MAX_POST, :]
    return (cls_pred, reg_pred, boxes)


def kernel(feats, ancs, ancs_valid, W_b, b_b, W_cls, b_cls, W_reg, b_reg):
    return _run(feats, ancs, ancs_valid, W_b, b_b, W_cls, b_cls, W_reg, b_reg)


# speedup vs baseline: 2.3267x; 1.5080x over previous
import functools

import jax
import jax.numpy as jnp
from jax import lax
from jax.experimental import pallas as pl
from jax.experimental.pallas import tpu as pltpu
from jax.experimental.pallas import tpu_sc as plsc

GH, GW, NA, CIN, CMID = 32, 32, 9, 768, 256
NPOS = GH * GW
N = NPOS * NA
BK = 128
NB = N // BK
MAX_POST = 2000
NOUT = 2048
NOB = NOUT // BK
IOU_T = 0.7
CW = 1024
SC_NW = 32
BPW = N // SC_NW


def _heads_kernel(x_ref, wb_ref, bb_ref, wc_ref, bc_ref, wr_ref, br_ref,
                  valid_ref, cls_ref, reg_ref, sm_ref):
    h = jnp.maximum(
        jnp.dot(x_ref[:], wb_ref[:], preferred_element_type=jnp.float32)
        + bb_ref[:], 0.0)
    logits = jnp.dot(h, wc_ref[:], preferred_element_type=jnp.float32) + bc_ref[:]
    cls = jax.nn.sigmoid(logits)
    reg = jnp.dot(h, wr_ref[:], preferred_element_type=jnp.float32) + br_ref[:]
    cls_ref[:] = cls
    reg_ref[:] = reg
    sm_ref[:] = jnp.where(valid_ref[:] > 0.0, cls, -jnp.inf)


def _rank_kernel(s_row_ref, s_col_ref, a4_ref, rank_ref, p16_ref, p4s):
    f32 = jnp.float32
    a0 = a4_ref[0:1, :]
    a1 = a4_ref[1:2, :]
    a2 = a4_ref[2:3, :]
    a3 = a4_ref[3:4, :]
    c0 = a0 + a0 * a2
    c1 = a1 + a1 * a3
    w0 = a2 * jnp.exp(a2)
    w1 = a3 * jnp.exp(a3)
    P0 = c0 - w0 * 0.5
    P1 = c1 - w1 * 0.5
    P2 = c0 + w0 * 0.5
    P3 = c1 + w1 * 0.5
    p4s[:] = jnp.concatenate([P0, P1, P2, P3], axis=0)

    iota_c = jax.lax.broadcasted_iota(jnp.int32, (BK, 1), 0)
    idx_row = jax.lax.broadcasted_iota(jnp.int32, (1, N), 1)
    s_row = s_row_ref[:]
    zpad = jnp.zeros((BK, 124), f32)

    def body(bi, carry):
        base = bi * BK
        sc = s_col_ref[pl.ds(base, BK), :]
        ic = iota_c + base
        gt = (s_row > sc).astype(f32)
        tie = ((s_row == sc) & (idx_row > ic)).astype(f32)
        rk = jnp.sum(gt + tie, axis=1, keepdims=True)
        rank_ref[pl.ds(base, BK), :] = rk.astype(jnp.int32)
        blk = jnp.transpose(p4s[:, pl.ds(base, BK)], (1, 0))
        p16_ref[pl.ds(base, BK), :] = jnp.concatenate([blk, zpad], axis=1)
        return carry

    jax.lax.fori_loop(0, NB, body, 0, unroll=False)


def _permute_sc(p16_hbm, rank_hbm, out_hbm, idx_v, rows_v, sem):
    wid = lax.axis_index("s") * 2 + lax.axis_index("c")
    base = wid * BPW
    pltpu.sync_copy(rank_hbm.at[pl.ds(base, BPW)], idx_v)
    pltpu.sync_copy(p16_hbm.at[pl.ds(base, BPW)], rows_v)
    pltpu.async_copy(rows_v, out_hbm.at[idx_v], sem).wait()


def _nms_kernel(sb_ref, out_ref, sP, keep_r, sup_r, destk_c):
    f32 = jnp.float32
    iota_c = jax.lax.broadcasted_iota(jnp.int32, (BK, 1), 0)
    iota_r = jax.lax.broadcasted_iota(jnp.int32, (1, BK), 1)
    strict_upper = (iota_c < iota_r).astype(f32)

    def plane_body(b, carry):
        base = b * BK
        t = jnp.transpose(sb_ref[pl.ds(base, BK), :], (1, 0))
        sP[:, pl.ds(base, BK)] = t[0:4, :]
        return carry

    jax.lax.fori_loop(0, NB, plane_body, 0, unroll=False)
    sup_r[:] = jnp.zeros((1, N + CW), f32)

    def nms_body(b, carry):
        base = b * BK
        y1c = sb_ref[pl.ds(base, BK), 0:1]
        x1c = sb_ref[pl.ds(base, BK), 1:2]
        y2c = sb_ref[pl.ds(base, BK), 2:3]
        x2c = sb_ref[pl.ds(base, BK), 3:4]
        ac = (y2c - y1c) * (x2c - x1c)
        y1r = sP[0:1, pl.ds(base, BK)]
        x1r = sP[1:2, pl.ds(base, BK)]
        y2r = sP[2:3, pl.ds(base, BK)]
        x2r = sP[3:4, pl.ds(base, BK)]
        ar = (y2r - y1r) * (x2r - x1r)
        supc = jnp.transpose(sup_r[0:1, pl.ds(base, BK)], (1, 0))
        ih = jnp.minimum(y2c, y2r) - jnp.maximum(y1c, y1r)
        iw = jnp.minimum(x2c, x2r) - jnp.maximum(x1c, x1r)
        inter = jnp.maximum(ih, 0.0) * jnp.maximum(iw, 0.0)
        Sb = (inter > IOU_T * (ac + ar - inter + 1e-9)).astype(f32) * strict_upper
        init_r = jnp.transpose((supc == 0.0).astype(f32), (1, 0))

        def fcond(st):
            it, changed, _ = st
            return (it < BK + 2) & changed

        def fbody(st):
            it, _, k = st
            su = jnp.dot(k, Sb, preferred_element_type=f32)
            nk = init_r * (su == 0.0).astype(f32)
            return it + 1, jnp.any(nk != k), nk

        _, _, k_row = jax.lax.while_loop(fcond, fbody, (0, True, init_r))
        keep_r[0:1, pl.ds(base, BK)] = k_row
        k_col = jnp.transpose(k_row, (1, 0))
        tac = ac + 1e-9

        def chunk_body(t, c):
            cb = base + BK + t * CW
            jy1 = sP[0:1, pl.ds(cb, CW)]
            jx1 = sP[1:2, pl.ds(cb, CW)]
            jy2 = sP[2:3, pl.ds(cb, CW)]
            jx2 = sP[3:4, pl.ds(cb, CW)]
            jar = (jy2 - jy1) * (jx2 - jx1)
            cih = jnp.minimum(y2c, jy2) - jnp.maximum(y1c, jy1)
            ciw = jnp.minimum(x2c, jx2) - jnp.maximum(x1c, jx1)
            cin_ = jnp.maximum(cih, 0.0) * jnp.maximum(ciw, 0.0)
            csup = (cin_ > IOU_T * (tac + jar - cin_)).astype(f32) * k_col
            news = (jnp.sum(csup, axis=0, keepdims=True) > 0.0).astype(f32)
            sup_r[0:1, pl.ds(cb, CW)] = jnp.maximum(sup_r[0:1, pl.ds(cb, CW)], news)
            return c

        ntrip = (N - base - BK + CW - 1) // CW
        jax.lax.fori_loop(0, ntrip, chunk_body, 0, unroll=False)
        return carry

    jax.lax.fori_loop(0, NB, nms_body, 0, unroll=False)

    def dest_body(b, nkept):
        krb = keep_r[0:1, pl.ds(b * BK, BK)]
        excl = jnp.dot(krb, strict_upper, preferred_element_type=f32)
        destk = jnp.where(krb > 0.0, excl + nkept, -1.0)
        destk_c[pl.ds(b * BK, BK), :] = jnp.transpose(destk, (1, 0))
        return nkept + jnp.sum(krb)

    nkept = jax.lax.fori_loop(0, NB, dest_body, 0.0, unroll=False)

    destk_all = destk_c[:]
    sb0 = sP[:, 0:1]

    def out_body(ob, carry):
        p = (iota_r + ob * BK).astype(f32)
        oh = (destk_all == p).astype(f32)
        blk = jnp.dot(sP[:, 0:N], oh, preferred_element_type=f32,
                      precision=jax.lax.Precision.HIGHEST)
        blk = blk + (p >= nkept).astype(f32) * sb0
        out_ref[:, pl.ds(ob * BK, BK)] = blk
        return carry

    jax.lax.fori_loop(0, NOB, out_body, 0, unroll=False)


@functools.partial(jax.jit, static_argnames=())
def _run(feats, ancs, ancs_valid, W_b, b_b, W_cls, b_cls, W_reg, b_reg):
    x = feats.reshape(NPOS, CIN)
    valid2d = ancs_valid.reshape(NPOS, NA)
    cls2d, reg2d, sm2d = pl.pallas_call(
        _heads_kernel,
        out_shape=(
            jax.ShapeDtypeStruct((NPOS, NA), jnp.float32),
            jax.ShapeDtypeStruct((NPOS, NA * 4), jnp.float32),
            jax.ShapeDtypeStruct((NPOS, NA), jnp.float32),
        ),
    )(x, W_b, b_b.reshape(1, CMID), W_cls, b_cls.reshape(1, NA),
      W_reg, b_reg.reshape(1, NA * 4), valid2d)

    s_row = sm2d.reshape(1, N)
    s_col = sm2d.reshape(N, 1)
    a4 = ancs.reshape(N, 4).T

    rank2d, p16 = pl.pallas_call(
        _rank_kernel,
        out_shape=(
            jax.ShapeDtypeStruct((N, 1), jnp.int32),
            jax.ShapeDtypeStruct((N, 128), jnp.float32),
        ),
        scratch_shapes=[pltpu.VMEM((4, N), jnp.float32)],
    )(s_row, s_col, a4)

    mesh = plsc.VectorSubcoreMesh(core_axis_name="c", subcore_axis_name="s")
    sorted16 = pl.kernel(
        _permute_sc,
        mesh=mesh,
        out_type=jax.ShapeDtypeStruct((N, 128), jnp.float32),
        scratch_types=[
            pltpu.VMEM((BPW,), jnp.int32),
            pltpu.VMEM((BPW, 128), jnp.float32),
            pltpu.SemaphoreType.DMA,
        ],
    )(p16, rank2d.reshape(N))

    out4 = pl.pallas_call(
        _nms_kernel,
        out_shape=jax.ShapeDtypeStruct((4, NOUT), jnp.float32),
        scratch_shapes=[
            pltpu.VMEM((4, N + CW), jnp.float32),
            pltpu.VMEM((1, N), jnp.float32),
            pltpu.VMEM((1, N + CW), jnp.float32),
            pltpu.VMEM((N, 1), jnp.float32),
        ],
    )(sorted16)

    cls_pred = cls2d.reshape(1, GH, GW, NA)
    reg_pred = reg2d.reshape(1, GH, GW, NA, 4)
    boxes = out4.T[:MAX_POST, :]
    return (cls_pred, reg_pred, boxes)


def kernel(feats, ancs, ancs_valid, W_b, b_b, W_cls, b_cls, W_reg, b_reg):
    return _run(feats, ancs, ancs_valid, W_b, b_b, W_cls, b_cls, W_reg, b_reg)


# speedup vs baseline: 2.3381x; 1.0049x over previous
import functools

import jax
import jax.numpy as jnp
from jax import lax
from jax.experimental import pallas as pl
from jax.experimental.pallas import tpu as pltpu
from jax.experimental.pallas import tpu_sc as plsc

GH, GW, NA, CIN, CMID = 32, 32, 9, 768, 256
NPOS = GH * GW
N = NPOS * NA
BK = 128
NB = N // BK
MAX_POST = 2000
NOUT = 2048
NOB = NOUT // BK
IOU_T = 0.7
CW = 1024
SC_NW = 32
BPW = N // SC_NW


def _heads_kernel(x_ref, wb_ref, bb_ref, wc_ref, bc_ref, wr_ref, br_ref,
                  valid_ref, cls_ref, reg_ref, sm_ref):
    h = jnp.maximum(
        jnp.dot(x_ref[:], wb_ref[:], preferred_element_type=jnp.float32)
        + bb_ref[:], 0.0)
    logits = jnp.dot(h, wc_ref[:], preferred_element_type=jnp.float32) + bc_ref[:]
    cls = jax.nn.sigmoid(logits)
    reg = jnp.dot(h, wr_ref[:], preferred_element_type=jnp.float32) + br_ref[:]
    cls_ref[:] = cls
    reg_ref[:] = reg
    sm_ref[:] = jnp.where(valid_ref[:] > 0.0, cls, -jnp.inf)


def _rank_kernel(s_row_ref, s_col_ref, a4_ref, rank_ref, p16_ref, p4s,
                 spad_r, aux_r):
    f32 = jnp.float32
    a0 = a4_ref[0:1, :]
    a1 = a4_ref[1:2, :]
    a2 = a4_ref[2:3, :]
    a3 = a4_ref[3:4, :]
    c0 = a0 + a0 * a2
    c1 = a1 + a1 * a3
    w0 = a2 * jnp.exp(a2)
    w1 = a3 * jnp.exp(a3)
    P0 = c0 - w0 * 0.5
    P1 = c1 - w1 * 0.5
    P2 = c0 + w0 * 0.5
    P3 = c1 + w1 * 0.5
    p4s[:] = jnp.concatenate([P0, P1, P2, P3], axis=0)

    iota_c = jax.lax.broadcasted_iota(jnp.int32, (BK, 1), 0)
    iota_r = jax.lax.broadcasted_iota(jnp.int32, (1, BK), 1)
    su_bool = iota_c < iota_r
    zpad = jnp.zeros((BK, 124), f32)
    spad_r[0:1, 0:N] = s_row_ref[:]
    spad_r[0:1, N:N + CW] = jnp.full((1, CW), -jnp.inf, f32)
    aux_r[:] = jnp.zeros((1, N + CW), f32)

    def body(bi, carry):
        base = bi * BK
        sc = s_col_ref[pl.ds(base, BK), :]
        srb = spad_r[0:1, pl.ds(base, BK)]
        diag = (srb > sc) | ((srb == sc) & su_bool)
        acc0 = jnp.sum(diag.astype(f32), axis=1, keepdims=True)

        def chunk(t, acc):
            cb = base + BK + t * CW
            sj = spad_r[0:1, pl.ds(cb, CW)]
            acc = acc + jnp.sum((sj >= sc).astype(f32), axis=1, keepdims=True)
            rev = jnp.sum((sj < sc).astype(f32), axis=0, keepdims=True)
            aux_r[0:1, pl.ds(cb, CW)] = aux_r[0:1, pl.ds(cb, CW)] + rev
            return acc

        ntrip = (N - base - BK + CW - 1) // CW
        acc = jax.lax.fori_loop(0, ntrip, chunk, acc0, unroll=False)
        rk = acc + jnp.transpose(aux_r[0:1, pl.ds(base, BK)], (1, 0))
        rank_ref[pl.ds(base, BK), :] = rk.astype(jnp.int32)
        blk = jnp.transpose(p4s[:, pl.ds(base, BK)], (1, 0))
        p16_ref[pl.ds(base, BK), :] = jnp.concatenate([blk, zpad], axis=1)
        return carry

    jax.lax.fori_loop(0, NB, body, 0, unroll=False)


def _permute_sc(p16_hbm, rank_hbm, out_hbm, idx_v, rows_v, sem):
    wid = lax.axis_index("s") * 2 + lax.axis_index("c")
    base = wid * BPW
    pltpu.sync_copy(rank_hbm.at[pl.ds(base, BPW)], idx_v)
    pltpu.sync_copy(p16_hbm.at[pl.ds(base, BPW)], rows_v)
    pltpu.async_copy(rows_v, out_hbm.at[idx_v], sem).wait()


def _nms_kernel(sb_ref, out_ref, sP, keep_r, sup_r, destk_c):
    f32 = jnp.float32
    iota_c = jax.lax.broadcasted_iota(jnp.int32, (BK, 1), 0)
    iota_r = jax.lax.broadcasted_iota(jnp.int32, (1, BK), 1)
    strict_upper = (iota_c < iota_r).astype(f32)

    def plane_body(b, carry):
        base = b * BK
        t = jnp.transpose(sb_ref[pl.ds(base, BK), :], (1, 0))
        sP[:, pl.ds(base, BK)] = t[0:4, :]
        return carry

    jax.lax.fori_loop(0, NB, plane_body, 0, unroll=False)
    sup_r[:] = jnp.zeros((1, N + CW), f32)

    def nms_body(b, carry):
        base = b * BK
        y1c = sb_ref[pl.ds(base, BK), 0:1]
        x1c = sb_ref[pl.ds(base, BK), 1:2]
        y2c = sb_ref[pl.ds(base, BK), 2:3]
        x2c = sb_ref[pl.ds(base, BK), 3:4]
        ac = (y2c - y1c) * (x2c - x1c)
        y1r = sP[0:1, pl.ds(base, BK)]
        x1r = sP[1:2, pl.ds(base, BK)]
        y2r = sP[2:3, pl.ds(base, BK)]
        x2r = sP[3:4, pl.ds(base, BK)]
        ar = (y2r - y1r) * (x2r - x1r)
        supc = jnp.transpose(sup_r[0:1, pl.ds(base, BK)], (1, 0))
        ih = jnp.minimum(y2c, y2r) - jnp.maximum(y1c, y1r)
        iw = jnp.minimum(x2c, x2r) - jnp.maximum(x1c, x1r)
        inter = jnp.maximum(ih, 0.0) * jnp.maximum(iw, 0.0)
        Sb = (inter > IOU_T * (ac + ar - inter + 1e-9)).astype(f32) * strict_upper
        init_r = jnp.transpose((supc == 0.0).astype(f32), (1, 0))

        def fcond(st):
            it, changed, _ = st
            return (it < BK + 2) & changed

        def fbody(st):
            it, _, k = st
            su = jnp.dot(k, Sb, preferred_element_type=f32)
            nk = init_r * (su == 0.0).astype(f32)
            return it + 1, jnp.any(nk != k), nk

        _, _, k_row = jax.lax.while_loop(fcond, fbody, (0, True, init_r))
        keep_r[0:1, pl.ds(base, BK)] = k_row
        k_col = jnp.transpose(k_row, (1, 0))
        tac = ac + 1e-9

        def chunk_body(t, c):
            cb = base + BK + t * CW
            jy1 = sP[0:1, pl.ds(cb, CW)]
            jx1 = sP[1:2, pl.ds(cb, CW)]
            jy2 = sP[2:3, pl.ds(cb, CW)]
            jx2 = sP[3:4, pl.ds(cb, CW)]
            jar = (jy2 - jy1) * (jx2 - jx1)
            cih = jnp.minimum(y2c, jy2) - jnp.maximum(y1c, jy1)
            ciw = jnp.minimum(x2c, jx2) - jnp.maximum(x1c, jx1)
            cin_ = jnp.maximum(cih, 0.0) * jnp.maximum(ciw, 0.0)
            csup = (cin_ > IOU_T * (tac + jar - cin_)).astype(f32) * k_col
            news = (jnp.sum(csup, axis=0, keepdims=True) > 0.0).astype(f32)
            sup_r[0:1, pl.ds(cb, CW)] = jnp.maximum(sup_r[0:1, pl.ds(cb, CW)], news)
            return c

        ntrip = (N - base - BK + CW - 1) // CW
        jax.lax.fori_loop(0, ntrip, chunk_body, 0, unroll=False)
        return carry

    jax.lax.fori_loop(0, NB, nms_body, 0, unroll=False)

    def dest_body(b, nkept):
        krb = keep_r[0:1, pl.ds(b * BK, BK)]
        excl = jnp.dot(krb, strict_upper, preferred_element_type=f32)
        destk = jnp.where(krb > 0.0, excl + nkept, -1.0)
        destk_c[pl.ds(b * BK, BK), :] = jnp.transpose(destk, (1, 0))
        return nkept + jnp.sum(krb)

    nkept = jax.lax.fori_loop(0, NB, dest_body, 0.0, unroll=False)

    destk_all = destk_c[:]
    sb0 = sP[:, 0:1]

    def out_body(ob, carry):
        p = (iota_r + ob * BK).astype(f32)
        oh = (destk_all == p).astype(f32)
        blk = jnp.dot(sP[:, 0:N], oh, preferred_element_type=f32,
                      precision=jax.lax.Precision.HIGHEST)
        blk = blk + (p >= nkept).astype(f32) * sb0
        out_ref[:, pl.ds(ob * BK, BK)] = blk
        return carry

    jax.lax.fori_loop(0, NOB, out_body, 0, unroll=False)


@functools.partial(jax.jit, static_argnames=())
def _run(feats, ancs, ancs_valid, W_b, b_b, W_cls, b_cls, W_reg, b_reg):
    x = feats.reshape(NPOS, CIN)
    valid2d = ancs_valid.reshape(NPOS, NA)
    cls2d, reg2d, sm2d = pl.pallas_call(
        _heads_kernel,
        out_shape=(
            jax.ShapeDtypeStruct((NPOS, NA), jnp.float32),
            jax.ShapeDtypeStruct((NPOS, NA * 4), jnp.float32),
            jax.ShapeDtypeStruct((NPOS, NA), jnp.float32),
        ),
    )(x, W_b, b_b.reshape(1, CMID), W_cls, b_cls.reshape(1, NA),
      W_reg, b_reg.reshape(1, NA * 4), valid2d)

    s_row = sm2d.reshape(1, N)
    s_col = sm2d.reshape(N, 1)
    a4 = ancs.reshape(N, 4).T

    rank2d, p16 = pl.pallas_call(
        _rank_kernel,
        out_shape=(
            jax.ShapeDtypeStruct((N, 1), jnp.int32),
            jax.ShapeDtypeStruct((N, 128), jnp.float32),
        ),
        scratch_shapes=[
            pltpu.VMEM((4, N), jnp.float32),
            pltpu.VMEM((1, N + CW), jnp.float32),
            pltpu.VMEM((1, N + CW), jnp.float32),
        ],
    )(s_row, s_col, a4)

    mesh = plsc.VectorSubcoreMesh(core_axis_name="c", subcore_axis_name="s")
    sorted16 = pl.kernel(
        _permute_sc,
        mesh=mesh,
        out_type=jax.ShapeDtypeStruct((N, 128), jnp.float32),
        scratch_types=[
            pltpu.VMEM((BPW,), jnp.int32),
            pltpu.VMEM((BPW, 128), jnp.float32),
            pltpu.SemaphoreType.DMA,
        ],
    )(p16, rank2d.reshape(N))

    out4 = pl.pallas_call(
        _nms_kernel,
        out_shape=jax.ShapeDtypeStruct((4, NOUT), jnp.float32),
        scratch_shapes=[
            pltpu.VMEM((4, N + CW), jnp.float32),
            pltpu.VMEM((1, N), jnp.float32),
            pltpu.VMEM((1, N + CW), jnp.float32),
            pltpu.VMEM((N, 1), jnp.float32),
        ],
    )(sorted16)

    cls_pred = cls2d.reshape(1, GH, GW, NA)
    reg_pred = reg2d.reshape(1, GH, GW, NA, 4)
    boxes = out4.T[:MAX_POST, :]
    return (cls_pred, reg_pred, boxes)


def kernel(feats, ancs, ancs_valid, W_b, b_b, W_cls, b_cls, W_reg, b_reg):
    return _run(feats, ancs, ancs_valid, W_b, b_b, W_cls, b_cls, W_reg, b_reg)
